# baseline (speedup 1.0000x reference)
import functools

import jax
import jax.numpy as jnp
from jax import lax
from jax.experimental import pallas as pl
from jax.experimental.pallas import tpu as pltpu
from jax.experimental.pallas import tpu_sc as plsc

B, S, D = 2, 2048, 1024
E, K, FF = 8, 2, 1024
N = B * S
T = 128
L = K * N + E * T
NT = L // T
TK1 = 512
GP = 128


def _gate_body(x_ref, gw_ref, gb_ref, gate_ref):
    xt = x_ref[...]
    logits = jnp.dot(xt, gw_ref[...], preferred_element_type=jnp.float32)
    logits = logits + gb_ref[...]
    m = jnp.max(logits, axis=-1, keepdims=True)
    p = jnp.exp(logits - m)
    p = p / jnp.sum(p, axis=-1, keepdims=True)
    idxs = jax.lax.broadcasted_iota(jnp.int32, (TK1, GP), 1)
    w0 = jnp.max(p, axis=-1, keepdims=True)
    i0 = jnp.min(jnp.where(p == w0, idxs, GP), axis=-1, keepdims=True)
    p2 = jnp.where(idxs == i0, -1.0, p)
    w1 = jnp.max(p2, axis=-1, keepdims=True)
    i1 = jnp.min(jnp.where(p2 == w1, idxs, GP), axis=-1, keepdims=True)
    out = jnp.where(idxs == 0, w0, 0.0)
    out = jnp.where(idxs == 1, w1, out)
    out = jnp.where(idxs == 2, i0.astype(jnp.float32), out)
    out = jnp.where(idxs == 3, i1.astype(jnp.float32), out)
    gate_ref[...] = out


def _gate(x2d, gwp, gbp):
    return pl.pallas_call(
        _gate_body,
        grid=(N // TK1,),
        in_specs=[
            pl.BlockSpec((TK1, D), lambda t: (t, 0)),
            pl.BlockSpec((D, GP), lambda t: (0, 0)),
            pl.BlockSpec((1, GP), lambda t: (0, 0)),
        ],
        out_specs=pl.BlockSpec((TK1, GP), lambda t: (t, 0)),
        out_shape=jax.ShapeDtypeStruct((N, GP), jnp.float32),
    )(x2d, gwp, gbp)


def _shared_body(x_ref, sw1_ref, sb1_ref, sw2_ref, sb2_ref, sh_ref):
    xt = x_ref[...]
    h = jnp.dot(xt, sw1_ref[...], preferred_element_type=jnp.float32)
    h = h + sb1_ref[...]
    a = h[:, :FF]
    b = h[:, FF:]
    g = a * jax.nn.sigmoid(a) * b
    sh = jnp.dot(g, sw2_ref[...], preferred_element_type=jnp.float32)
    sh_ref[...] = sh + sb2_ref[...]


def _shared_ffn(x2d, sw1, sb1, sw2, sb2):
    return pl.pallas_call(
        _shared_body,
        grid=(N // TK1,),
        in_specs=[
            pl.BlockSpec((TK1, D), lambda t: (t, 0)),
            pl.BlockSpec((D, 2 * FF), lambda t: (0, 0)),
            pl.BlockSpec((1, 2 * FF), lambda t: (0, 0)),
            pl.BlockSpec((FF, D), lambda t: (0, 0)),
            pl.BlockSpec((1, D), lambda t: (0, 0)),
        ],
        out_specs=pl.BlockSpec((TK1, D), lambda t: (t, 0)),
        out_shape=jax.ShapeDtypeStruct((N, D), jnp.float32),
    )(x2d, sw1, sb1, sw2, sb2)


def _k2_body(te_ref, xd_ref, w1_ref, b1_ref, w2_ref, b2_ref, y_ref):
    xt = xd_ref[...]
    h = jnp.dot(xt, w1_ref[0], preferred_element_type=jnp.float32)
    h = h + b1_ref[0]
    a = h[:, :FF]
    b = h[:, FF:]
    g = a * jax.nn.sigmoid(a) * b
    y = jnp.dot(g, w2_ref[0], preferred_element_type=jnp.float32)
    y_ref[...] = y + b2_ref[0]


def _grouped_ffn(xd, rw1, rb1, rw2, rb2, tile_expert):
    grid_spec = pltpu.PrefetchScalarGridSpec(
        num_scalar_prefetch=1,
        grid=(NT,),
        in_specs=[
            pl.BlockSpec((T, D), lambda t, te: (t, 0)),
            pl.BlockSpec((1, D, 2 * FF), lambda t, te: (te[t], 0, 0)),
            pl.BlockSpec((1, 1, 2 * FF), lambda t, te: (te[t], 0, 0)),
            pl.BlockSpec((1, FF, D), lambda t, te: (te[t], 0, 0)),
            pl.BlockSpec((1, 1, D), lambda t, te: (te[t], 0, 0)),
        ],
        out_specs=pl.BlockSpec((T, D), lambda t, te: (t, 0)),
    )
    return pl.pallas_call(
        _k2_body,
        grid_spec=grid_spec,
        out_shape=jax.ShapeDtypeStruct((L, D), jnp.float32),
    )(tile_expert, xd, rw1, rb1.reshape(E, 1, 2 * FF), rw2,
      rb2.reshape(E, 1, D))


_NC, _NS = 2, 16
_NW = _NC * _NS
_TPW = N // _NW
_DCH = 32
_DNCH = _TPW // _DCH
_CCH = 16
_CNCH = _TPW // _CCH


def _sc_dispatch_body(x_hbm, p0_hbm, p1_hbm, xd_hbm,
                      i0_v, i1_v, r0, r1, l0s, l1s, s0s, s1s):
    wid = lax.axis_index("s") * _NC + lax.axis_index("c")
    base = wid * _TPW
    for c in range(_DNCH):
        pltpu.sync_copy(p0_hbm.at[pl.ds(base + c * _DCH, _DCH)], i0_v.at[c])
        pltpu.sync_copy(p1_hbm.at[pl.ds(base + c * _DCH, _DCH)], i1_v.at[c])
    bufs = (r0, r1)
    lsems = (l0s, l1s)
    ssems = (s0s, s1s)

    def start_load(c):
        return pltpu.async_copy(
            x_hbm.at[pl.ds(base + c * _DCH, _DCH)], bufs[c % 2],
            lsems[c % 2])

    loads = [None] * _DNCH
    sc0 = [None] * _DNCH
    sc1 = [None] * _DNCH
    loads[0] = start_load(0)
    if _DNCH > 1:
        loads[1] = start_load(1)
    for c in range(_DNCH):
        b = c % 2
        loads[c].wait()
        sc0[c] = pltpu.async_copy(bufs[b], xd_hbm.at[i0_v.at[c]], ssems[b])
        sc1[c] = pltpu.async_copy(bufs[b], xd_hbm.at[i1_v.at[c]], ssems[b])
        nxt = c + 2
        if nxt < _DNCH:
            sc0[c].wait()
            sc1[c].wait()
            loads[nxt] = start_load(nxt)
    for c in range(max(0, _DNCH - 2), _DNCH):
        sc0[c].wait()
        sc1[c].wait()


@functools.lru_cache(maxsize=None)
def _sc_dispatch_kernel():
    return pl.kernel(
        _sc_dispatch_body,
        out_type=jax.ShapeDtypeStruct((L, D), jnp.float32),
        mesh=plsc.VectorSubcoreMesh(core_axis_name="c", subcore_axis_name="s"),
        scratch_types=(
            [pltpu.VMEM((_DNCH, _DCH), jnp.int32),
             pltpu.VMEM((_DNCH, _DCH), jnp.int32)]
            + [pltpu.VMEM((_DCH, D), jnp.float32)] * 2
            + [pltpu.SemaphoreType.DMA] * 4
        ),
    )


def _sc_dispatch(x2d, pos0, pos1):
    return _sc_dispatch_kernel()(x2d, pos0, pos1)


def _sc_combine_body(sh_hbm, y_hbm, p0_hbm, p1_hbm, w0_hbm, w1_hbm, out_hbm,
                     i0_v, i1_v, w0_v, w1_v, g0a, g1a, sha, g0b, g1b, shb,
                     sg0a, sg1a, ssha, sg0b, sg1b, sshb, sta, stb):
    wid = lax.axis_index("s") * _NC + lax.axis_index("c")
    base = wid * _TPW
    pltpu.sync_copy(p0_hbm.at[pl.ds(base, _TPW)], i0_v)
    pltpu.sync_copy(p1_hbm.at[pl.ds(base, _TPW)], i1_v)
    pltpu.sync_copy(w0_hbm.at[pl.ds(base, _TPW)], w0_v)
    pltpu.sync_copy(w1_hbm.at[pl.ds(base, _TPW)], w1_v)
    g0 = (g0a, g0b)
    g1 = (g1a, g1b)
    sh = (sha, shb)
    sg0 = (sg0a, sg0b)
    sg1 = (sg1a, sg1b)
    ssh = (ssha, sshb)
    st = (sta, stb)

    def start_in(c):
        b = c % 2
        d1 = pltpu.async_copy(
            y_hbm.at[i0_v.at[pl.ds(c * _CCH, _CCH)]], g0[b], sg0[b])
        d2 = pltpu.async_copy(
            y_hbm.at[i1_v.at[pl.ds(c * _CCH, _CCH)]], g1[b], sg1[b])
        d3 = pltpu.async_copy(
            sh_hbm.at[pl.ds(base + c * _CCH, _CCH)], sh[b], ssh[b])
        return (d1, d2, d3)

    ins = [None] * _CNCH
    outs = [None] * _CNCH
    ins[0] = start_in(0)
    for c in range(_CNCH):
        b = c % 2
        for dsc in ins[c]:
            dsc.wait()
        if c + 1 < _CNCH:
            if c >= 1:
                outs[c - 1].wait()
            ins[c + 1] = start_in(c + 1)

        wv0 = w0_v[pl.ds(c * _CCH, _CCH)]
        wv1 = w1_v[pl.ds(c * _CCH, _CCH)]

        def col(jj, carry, _b=b, _w0=wv0, _w1=wv1):
            sl = pl.ds(jj * 16, 16)
            for r in range(_CCH):
                sh[_b][r, sl] = (sh[_b][r, sl] + _w0[r] * g0[_b][r, sl]
                                 + _w1[r] * g1[_b][r, sl])
            return carry

        lax.fori_loop(0, D // 16, col, 0)
        outs[c] = pltpu.async_copy(
            sh[b], out_hbm.at[pl.ds(base + c * _CCH, _CCH)], st[b])
    outs[_CNCH - 2].wait()
    outs[_CNCH - 1].wait()


@functools.lru_cache(maxsize=None)
def _sc_combine_kernel():
    return pl.kernel(
        _sc_combine_body,
        out_type=jax.ShapeDtypeStruct((N, D), jnp.float32),
        mesh=plsc.VectorSubcoreMesh(core_axis_name="c", subcore_axis_name="s"),
        scratch_types=(
            [pltpu.VMEM((_TPW,), jnp.int32),
             pltpu.VMEM((_TPW,), jnp.int32),
             pltpu.VMEM((_TPW,), jnp.float32),
             pltpu.VMEM((_TPW,), jnp.float32)]
            + [pltpu.VMEM((_CCH, D), jnp.float32)] * 6
            + [pltpu.SemaphoreType.DMA] * 8
        ),
    )


def _sc_combine(shared2d, y, pos0, pos1, w0, w1):
    return _sc_combine_kernel()(shared2d, y, pos0, pos1, w0, w1)


def kernel(x, sw1, sb1, sw2, sb2, rw1, rb1, rw2, rb2, gw, gb):
    x2d = x.reshape(N, D)
    gwp = jnp.pad(gw, ((0, 0), (0, GP - E)))
    gbp = jnp.pad(gb, (0, GP - E), constant_values=-1e30).reshape(1, GP)

    gate = _gate(x2d, gwp, gbp)
    w0 = gate[:, 0]
    w1 = gate[:, 1]
    i0 = gate[:, 2].astype(jnp.int32)
    i1 = gate[:, 3].astype(jnp.int32)

    e_flat = jnp.concatenate([i0, i1])
    oh = (e_flat[:, None] == jnp.arange(E, dtype=jnp.int32)[None, :]
          ).astype(jnp.float32)
    BK = 128
    NB = (K * N) // BK
    oh3 = oh.reshape(NB, BK, E)
    tri = jnp.tril(jnp.ones((BK, BK), jnp.float32))
    intra = jax.lax.dot_general(
        oh3, tri, (((1,), (1,)), ((), ())),
        preferred_element_type=jnp.float32).transpose(0, 2, 1)
    blk_tot = intra[:, -1, :]
    tri_x = jnp.tril(jnp.ones((NB, NB), jnp.float32), -1)
    blk_pref = jax.lax.dot_general(
        tri_x, blk_tot, (((1,), (0,)), ((), ())),
        preferred_element_type=jnp.float32)
    prefix = (intra + blk_pref[:, None, :]).reshape(K * N, E)
    seq = (jnp.sum(oh * prefix, axis=1) - 1.0).astype(jnp.int32)
    counts = prefix[-1].astype(jnp.int32)
    tiles_per_e = (counts + T - 1) // T
    tile_start = jnp.concatenate(
        [jnp.zeros((1,), jnp.int32),
         jnp.cumsum(tiles_per_e)[:-1].astype(jnp.int32)])
    start = tile_start * T
    pos = (jnp.sum(oh * start[None, :].astype(jnp.float32), axis=1)
           ).astype(jnp.int32) + seq
    pos0, pos1 = pos[:N], pos[N:]
    tile_expert = (jnp.searchsorted(
        tile_start, jnp.arange(NT, dtype=jnp.int32), side='right') - 1
    ).astype(jnp.int32)

    xd = _sc_dispatch(x2d, pos0, pos1)
    shared2d = _shared_ffn(
        x2d, sw1, sb1.reshape(1, 2 * FF), sw2, sb2.reshape(1, D))

    y = _grouped_ffn(xd, rw1, rb1, rw2, rb2, tile_expert)

    out2d = _sc_combine(shared2d, y, pos0, pos1, w0, w1)
    return out2d.reshape(B, S, D)


# speedup vs baseline: 1.0499x; 1.0499x over previous
import functools

import jax
import jax.numpy as jnp
from jax import lax
from jax.experimental import pallas as pl
from jax.experimental.pallas import tpu as pltpu
from jax.experimental.pallas import tpu_sc as plsc

B, S, D = 2, 2048, 1024
E, K, FF = 8, 2, 1024
N = B * S
T = 256
L = K * N + E * T
NT = L // T
TK1 = 512
GP = 128


def _gate_body(x_ref, gw_ref, gb_ref, gate_ref):
    xt = x_ref[...]
    logits = jnp.dot(xt, gw_ref[...], preferred_element_type=jnp.float32)
    logits = logits + gb_ref[...]
    m = jnp.max(logits, axis=-1, keepdims=True)
    p = jnp.exp(logits - m)
    p = p / jnp.sum(p, axis=-1, keepdims=True)
    idxs = jax.lax.broadcasted_iota(jnp.int32, (TK1, GP), 1)
    w0 = jnp.max(p, axis=-1, keepdims=True)
    i0 = jnp.min(jnp.where(p == w0, idxs, GP), axis=-1, keepdims=True)
    p2 = jnp.where(idxs == i0, -1.0, p)
    w1 = jnp.max(p2, axis=-1, keepdims=True)
    i1 = jnp.min(jnp.where(p2 == w1, idxs, GP), axis=-1, keepdims=True)
    out = jnp.where(idxs == 0, w0, 0.0)
    out = jnp.where(idxs == 1, w1, out)
    out = jnp.where(idxs == 2, i0.astype(jnp.float32), out)
    out = jnp.where(idxs == 3, i1.astype(jnp.float32), out)
    gate_ref[...] = out


def _gate(x2d, gwp, gbp):
    return pl.pallas_call(
        _gate_body,
        grid=(N // TK1,),
        in_specs=[
            pl.BlockSpec((TK1, D), lambda t: (t, 0)),
            pl.BlockSpec((D, GP), lambda t: (0, 0)),
            pl.BlockSpec((1, GP), lambda t: (0, 0)),
        ],
        out_specs=pl.BlockSpec((TK1, GP), lambda t: (t, 0)),
        out_shape=jax.ShapeDtypeStruct((N, GP), jnp.float32),
    )(x2d, gwp, gbp)


def _shared_body(x_ref, sw1_ref, sb1_ref, sw2_ref, sb2_ref, sh_ref):
    xt = x_ref[...]
    h = jnp.dot(xt, sw1_ref[...], preferred_element_type=jnp.float32)
    h = h + sb1_ref[...]
    a = h[:, :FF]
    b = h[:, FF:]
    g = a * jax.nn.sigmoid(a) * b
    sh = jnp.dot(g, sw2_ref[...], preferred_element_type=jnp.float32)
    sh_ref[...] = sh + sb2_ref[...]


def _shared_ffn(x2d, sw1, sb1, sw2, sb2):
    return pl.pallas_call(
        _shared_body,
        grid=(N // TK1,),
        in_specs=[
            pl.BlockSpec((TK1, D), lambda t: (t, 0)),
            pl.BlockSpec((D, 2 * FF), lambda t: (0, 0)),
            pl.BlockSpec((1, 2 * FF), lambda t: (0, 0)),
            pl.BlockSpec((FF, D), lambda t: (0, 0)),
            pl.BlockSpec((1, D), lambda t: (0, 0)),
        ],
        out_specs=pl.BlockSpec((TK1, D), lambda t: (t, 0)),
        out_shape=jax.ShapeDtypeStruct((N, D), jnp.float32),
    )(x2d, sw1, sb1, sw2, sb2)


def _k2_body(te_ref, xd_ref, w1_ref, b1_ref, w2_ref, b2_ref, y_ref):
    xt = xd_ref[...].astype(jnp.bfloat16)
    h = jnp.dot(xt, w1_ref[0].astype(jnp.bfloat16),
                preferred_element_type=jnp.float32)
    h = h + b1_ref[0]
    a = h[:, :FF]
    b = h[:, FF:]
    g = a * jax.nn.sigmoid(a) * b
    y = jnp.dot(g.astype(jnp.bfloat16), w2_ref[0].astype(jnp.bfloat16),
                preferred_element_type=jnp.float32)
    y_ref[...] = y + b2_ref[0]


def _grouped_ffn(xd, rw1, rb1, rw2, rb2, tile_expert):
    grid_spec = pltpu.PrefetchScalarGridSpec(
        num_scalar_prefetch=1,
        grid=(NT,),
        in_specs=[
            pl.BlockSpec((T, D), lambda t, te: (t, 0)),
            pl.BlockSpec((1, D, 2 * FF), lambda t, te: (te[t], 0, 0)),
            pl.BlockSpec((1, 1, 2 * FF), lambda t, te: (te[t], 0, 0)),
            pl.BlockSpec((1, FF, D), lambda t, te: (te[t], 0, 0)),
            pl.BlockSpec((1, 1, D), lambda t, te: (te[t], 0, 0)),
        ],
        out_specs=pl.BlockSpec((T, D), lambda t, te: (t, 0)),
    )
    return pl.pallas_call(
        _k2_body,
        grid_spec=grid_spec,
        out_shape=jax.ShapeDtypeStruct((L, D), jnp.float32),
    )(tile_expert, xd, rw1, rb1.reshape(E, 1, 2 * FF), rw2,
      rb2.reshape(E, 1, D))


_NC, _NS = 2, 16
_NW = _NC * _NS
_TPW = N // _NW
_DCH = 32
_DNCH = _TPW // _DCH
_CCH = 16
_CNCH = _TPW // _CCH


def _sc_dispatch_body(x_hbm, p0_hbm, p1_hbm, xd_hbm,
                      i0_v, i1_v, r0, r1, l0s, l1s, s0s, s1s):
    wid = lax.axis_index("s") * _NC + lax.axis_index("c")
    base = wid * _TPW
    for c in range(_DNCH):
        pltpu.sync_copy(p0_hbm.at[pl.ds(base + c * _DCH, _DCH)], i0_v.at[c])
        pltpu.sync_copy(p1_hbm.at[pl.ds(base + c * _DCH, _DCH)], i1_v.at[c])
    bufs = (r0, r1)
    lsems = (l0s, l1s)
    ssems = (s0s, s1s)

    def start_load(c):
        return pltpu.async_copy(
            x_hbm.at[pl.ds(base + c * _DCH, _DCH)], bufs[c % 2],
            lsems[c % 2])

    loads = [None] * _DNCH
    sc0 = [None] * _DNCH
    sc1 = [None] * _DNCH
    loads[0] = start_load(0)
    if _DNCH > 1:
        loads[1] = start_load(1)
    for c in range(_DNCH):
        b = c % 2
        loads[c].wait()
        sc0[c] = pltpu.async_copy(bufs[b], xd_hbm.at[i0_v.at[c]], ssems[b])
        sc1[c] = pltpu.async_copy(bufs[b], xd_hbm.at[i1_v.at[c]], ssems[b])
        nxt = c + 2
        if nxt < _DNCH:
            sc0[c].wait()
            sc1[c].wait()
            loads[nxt] = start_load(nxt)
    for c in range(max(0, _DNCH - 2), _DNCH):
        sc0[c].wait()
        sc1[c].wait()


@functools.lru_cache(maxsize=None)
def _sc_dispatch_kernel():
    return pl.kernel(
        _sc_dispatch_body,
        out_type=jax.ShapeDtypeStruct((L, D), jnp.float32),
        mesh=plsc.VectorSubcoreMesh(core_axis_name="c", subcore_axis_name="s"),
        scratch_types=(
            [pltpu.VMEM((_DNCH, _DCH), jnp.int32),
             pltpu.VMEM((_DNCH, _DCH), jnp.int32)]
            + [pltpu.VMEM((_DCH, D), jnp.float32)] * 2
            + [pltpu.SemaphoreType.DMA] * 4
        ),
    )


def _sc_dispatch(x2d, pos0, pos1):
    return _sc_dispatch_kernel()(x2d, pos0, pos1)


def _sc_combine_body(sh_hbm, y_hbm, p0_hbm, p1_hbm, w0_hbm, w1_hbm, out_hbm,
                     i0_v, i1_v, w0_v, w1_v, g0a, g1a, sha, g0b, g1b, shb,
                     sg0a, sg1a, ssha, sg0b, sg1b, sshb, sta, stb):
    wid = lax.axis_index("s") * _NC + lax.axis_index("c")
    base = wid * _TPW
    pltpu.sync_copy(p0_hbm.at[pl.ds(base, _TPW)], i0_v)
    pltpu.sync_copy(p1_hbm.at[pl.ds(base, _TPW)], i1_v)
    pltpu.sync_copy(w0_hbm.at[pl.ds(base, _TPW)], w0_v)
    pltpu.sync_copy(w1_hbm.at[pl.ds(base, _TPW)], w1_v)
    g0 = (g0a, g0b)
    g1 = (g1a, g1b)
    sh = (sha, shb)
    sg0 = (sg0a, sg0b)
    sg1 = (sg1a, sg1b)
    ssh = (ssha, sshb)
    st = (sta, stb)

    def start_in(c):
        b = c % 2
        d1 = pltpu.async_copy(
            y_hbm.at[i0_v.at[pl.ds(c * _CCH, _CCH)]], g0[b], sg0[b])
        d2 = pltpu.async_copy(
            y_hbm.at[i1_v.at[pl.ds(c * _CCH, _CCH)]], g1[b], sg1[b])
        d3 = pltpu.async_copy(
            sh_hbm.at[pl.ds(base + c * _CCH, _CCH)], sh[b], ssh[b])
        return (d1, d2, d3)

    ins = [None] * _CNCH
    outs = [None] * _CNCH
    ins[0] = start_in(0)
    for c in range(_CNCH):
        b = c % 2
        for dsc in ins[c]:
            dsc.wait()
        if c + 1 < _CNCH:
            if c >= 1:
                outs[c - 1].wait()
            ins[c + 1] = start_in(c + 1)

        wv0 = w0_v[pl.ds(c * _CCH, _CCH)]
        wv1 = w1_v[pl.ds(c * _CCH, _CCH)]

        def col(jj, carry, _b=b, _w0=wv0, _w1=wv1):
            sl = pl.ds(jj * 16, 16)
            for r in range(_CCH):
                sh[_b][r, sl] = (sh[_b][r, sl] + _w0[r] * g0[_b][r, sl]
                                 + _w1[r] * g1[_b][r, sl])
            return carry

        lax.fori_loop(0, D // 16, col, 0)
        outs[c] = pltpu.async_copy(
            sh[b], out_hbm.at[pl.ds(base + c * _CCH, _CCH)], st[b])
    outs[_CNCH - 2].wait()
    outs[_CNCH - 1].wait()


@functools.lru_cache(maxsize=None)
def _sc_combine_kernel():
    return pl.kernel(
        _sc_combine_body,
        out_type=jax.ShapeDtypeStruct((N, D), jnp.float32),
        mesh=plsc.VectorSubcoreMesh(core_axis_name="c", subcore_axis_name="s"),
        scratch_types=(
            [pltpu.VMEM((_TPW,), jnp.int32),
             pltpu.VMEM((_TPW,), jnp.int32),
             pltpu.VMEM((_TPW,), jnp.float32),
             pltpu.VMEM((_TPW,), jnp.float32)]
            + [pltpu.VMEM((_CCH, D), jnp.float32)] * 6
            + [pltpu.SemaphoreType.DMA] * 8
        ),
    )


def _sc_combine(shared2d, y, pos0, pos1, w0, w1):
    return _sc_combine_kernel()(shared2d, y, pos0, pos1, w0, w1)


def kernel(x, sw1, sb1, sw2, sb2, rw1, rb1, rw2, rb2, gw, gb):
    x2d = x.reshape(N, D)
    gwp = jnp.pad(gw, ((0, 0), (0, GP - E)))
    gbp = jnp.pad(gb, (0, GP - E), constant_values=-1e30).reshape(1, GP)

    gate = _gate(x2d, gwp, gbp)
    w0 = gate[:, 0]
    w1 = gate[:, 1]
    i0 = gate[:, 2].astype(jnp.int32)
    i1 = gate[:, 3].astype(jnp.int32)

    e_flat = jnp.concatenate([i0, i1])
    oh = (e_flat[:, None] == jnp.arange(E, dtype=jnp.int32)[None, :]
          ).astype(jnp.float32)
    BK = 128
    NB = (K * N) // BK
    oh3 = oh.reshape(NB, BK, E)
    tri = jnp.tril(jnp.ones((BK, BK), jnp.float32))
    intra = jax.lax.dot_general(
        oh3, tri, (((1,), (1,)), ((), ())),
        preferred_element_type=jnp.float32).transpose(0, 2, 1)
    blk_tot = intra[:, -1, :]
    tri_x = jnp.tril(jnp.ones((NB, NB), jnp.float32), -1)
    blk_pref = jax.lax.dot_general(
        tri_x, blk_tot, (((1,), (0,)), ((), ())),
        preferred_element_type=jnp.float32)
    prefix = (intra + blk_pref[:, None, :]).reshape(K * N, E)
    seq = (jnp.sum(oh * prefix, axis=1) - 1.0).astype(jnp.int32)
    counts = prefix[-1].astype(jnp.int32)
    tiles_per_e = (counts + T - 1) // T
    tile_start = jnp.concatenate(
        [jnp.zeros((1,), jnp.int32),
         jnp.cumsum(tiles_per_e)[:-1].astype(jnp.int32)])
    start = tile_start * T
    pos = (jnp.sum(oh * start[None, :].astype(jnp.float32), axis=1)
           ).astype(jnp.int32) + seq
    pos0, pos1 = pos[:N], pos[N:]
    tile_expert = (jnp.searchsorted(
        tile_start, jnp.arange(NT, dtype=jnp.int32), side='right') - 1
    ).astype(jnp.int32)

    xd = _sc_dispatch(x2d, pos0, pos1)
    shared2d = _shared_ffn(
        x2d, sw1, sb1.reshape(1, 2 * FF), sw2, sb2.reshape(1, D))

    y = _grouped_ffn(xd, rw1, rb1, rw2, rb2, tile_expert)

    out2d = _sc_combine(shared2d, y, pos0, pos1, w0, w1)
    return out2d.reshape(B, S, D)


# speedup vs baseline: 1.1582x; 1.1031x over previous
import functools

import jax
import jax.numpy as jnp
from jax import lax
from jax.experimental import pallas as pl
from jax.experimental.pallas import tpu as pltpu
from jax.experimental.pallas import tpu_sc as plsc

B, S, D = 2, 2048, 1024
E, K, FF = 8, 2, 1024
N = B * S
T = 256
L = K * N + E * T
NT = L // T
TK1 = 512
GP = 128


def _gate_body(x_ref, gw_ref, gb_ref, gate_ref):
    xt = x_ref[...]
    logits = jnp.dot(xt, gw_ref[...], preferred_element_type=jnp.float32)
    logits = logits + gb_ref[...]
    m = jnp.max(logits, axis=-1, keepdims=True)
    p = jnp.exp(logits - m)
    p = p / jnp.sum(p, axis=-1, keepdims=True)
    idxs = jax.lax.broadcasted_iota(jnp.int32, (TK1, GP), 1)
    w0 = jnp.max(p, axis=-1, keepdims=True)
    i0 = jnp.min(jnp.where(p == w0, idxs, GP), axis=-1, keepdims=True)
    p2 = jnp.where(idxs == i0, -1.0, p)
    w1 = jnp.max(p2, axis=-1, keepdims=True)
    i1 = jnp.min(jnp.where(p2 == w1, idxs, GP), axis=-1, keepdims=True)
    out = jnp.where(idxs == 0, w0, 0.0)
    out = jnp.where(idxs == 1, w1, out)
    out = jnp.where(idxs == 2, i0.astype(jnp.float32), out)
    out = jnp.where(idxs == 3, i1.astype(jnp.float32), out)
    gate_ref[...] = out


def _gate(x2d, gwp, gbp):
    return pl.pallas_call(
        _gate_body,
        grid=(N // TK1,),
        in_specs=[
            pl.BlockSpec((TK1, D), lambda t: (t, 0)),
            pl.BlockSpec((D, GP), lambda t: (0, 0)),
            pl.BlockSpec((1, GP), lambda t: (0, 0)),
        ],
        out_specs=pl.BlockSpec((TK1, GP), lambda t: (t, 0)),
        out_shape=jax.ShapeDtypeStruct((N, GP), jnp.float32),
    )(x2d, gwp, gbp)


def _shared_body(x_ref, sw1_ref, sb1_ref, sw2_ref, sb2_ref, sh_ref):
    xt = x_ref[...]
    h = jnp.dot(xt, sw1_ref[...], preferred_element_type=jnp.float32)
    h = h + sb1_ref[...]
    a = h[:, :FF]
    b = h[:, FF:]
    g = a * jax.nn.sigmoid(a) * b
    sh = jnp.dot(g, sw2_ref[...], preferred_element_type=jnp.float32)
    sh_ref[...] = sh + sb2_ref[...]


def _shared_ffn(x2d, sw1, sb1, sw2, sb2):
    return pl.pallas_call(
        _shared_body,
        grid=(N // TK1,),
        in_specs=[
            pl.BlockSpec((TK1, D), lambda t: (t, 0)),
            pl.BlockSpec((D, 2 * FF), lambda t: (0, 0)),
            pl.BlockSpec((1, 2 * FF), lambda t: (0, 0)),
            pl.BlockSpec((FF, D), lambda t: (0, 0)),
            pl.BlockSpec((1, D), lambda t: (0, 0)),
        ],
        out_specs=pl.BlockSpec((TK1, D), lambda t: (t, 0)),
        out_shape=jax.ShapeDtypeStruct((N, D), jnp.float32),
    )(x2d, sw1, sb1, sw2, sb2)


def _k2_body(te_ref, xd_ref, w1_ref, b1_ref, w2_ref, b2_ref, y_ref):
    xt = xd_ref[...].astype(jnp.bfloat16)
    h = jnp.dot(xt, w1_ref[0].astype(jnp.bfloat16),
                preferred_element_type=jnp.float32)
    h = h + b1_ref[0]
    a = h[:, :FF]
    b = h[:, FF:]
    g = a * jax.nn.sigmoid(a) * b
    y = jnp.dot(g.astype(jnp.bfloat16), w2_ref[0].astype(jnp.bfloat16),
                preferred_element_type=jnp.float32)
    y_ref[...] = y + b2_ref[0]


def _grouped_ffn(xd, rw1, rb1, rw2, rb2, tile_expert):
    grid_spec = pltpu.PrefetchScalarGridSpec(
        num_scalar_prefetch=1,
        grid=(NT,),
        in_specs=[
            pl.BlockSpec((T, D), lambda t, te: (t, 0)),
            pl.BlockSpec((1, D, 2 * FF), lambda t, te: (te[t], 0, 0)),
            pl.BlockSpec((1, 1, 2 * FF), lambda t, te: (te[t], 0, 0)),
            pl.BlockSpec((1, FF, D), lambda t, te: (te[t], 0, 0)),
            pl.BlockSpec((1, 1, D), lambda t, te: (te[t], 0, 0)),
        ],
        out_specs=pl.BlockSpec((T, D), lambda t, te: (t, 0)),
    )
    return pl.pallas_call(
        _k2_body,
        grid_spec=grid_spec,
        out_shape=jax.ShapeDtypeStruct((L, D), jnp.float32),
    )(tile_expert, xd, rw1, rb1.reshape(E, 1, 2 * FF), rw2,
      rb2.reshape(E, 1, D))


_NC, _NS = 2, 16
_NW = _NC * _NS
_TPW = N // _NW
_DCH = 32
_DNCH = _TPW // _DCH
_CCH = 16
_CNCH = _TPW // _CCH


def _sc_dispatch_body(x_hbm, p0_hbm, p1_hbm, xd_hbm,
                      i0_v, i1_v, r0, r1, l0s, l1s, s0s, s1s):
    wid = lax.axis_index("s") * _NC + lax.axis_index("c")
    base = wid * _TPW
    for c in range(_DNCH):
        pltpu.sync_copy(p0_hbm.at[pl.ds(base + c * _DCH, _DCH)], i0_v.at[c])
        pltpu.sync_copy(p1_hbm.at[pl.ds(base + c * _DCH, _DCH)], i1_v.at[c])
    bufs = (r0, r1)
    lsems = (l0s, l1s)
    ssems = (s0s, s1s)

    def start_load(c):
        return pltpu.async_copy(
            x_hbm.at[pl.ds(base + c * _DCH, _DCH)], bufs[c % 2],
            lsems[c % 2])

    loads = [None] * _DNCH
    sc0 = [None] * _DNCH
    sc1 = [None] * _DNCH
    loads[0] = start_load(0)
    if _DNCH > 1:
        loads[1] = start_load(1)
    for c in range(_DNCH):
        b = c % 2
        loads[c].wait()
        sc0[c] = pltpu.async_copy(bufs[b], xd_hbm.at[i0_v.at[c]], ssems[b])
        sc1[c] = pltpu.async_copy(bufs[b], xd_hbm.at[i1_v.at[c]], ssems[b])
        nxt = c + 2
        if nxt < _DNCH:
            sc0[c].wait()
            sc1[c].wait()
            loads[nxt] = start_load(nxt)
    for c in range(max(0, _DNCH - 2), _DNCH):
        sc0[c].wait()
        sc1[c].wait()


@functools.lru_cache(maxsize=None)
def _sc_dispatch_kernel():
    return pl.kernel(
        _sc_dispatch_body,
        out_type=jax.ShapeDtypeStruct((L, D), jnp.float32),
        mesh=plsc.VectorSubcoreMesh(core_axis_name="c", subcore_axis_name="s"),
        scratch_types=(
            [pltpu.VMEM((_DNCH, _DCH), jnp.int32),
             pltpu.VMEM((_DNCH, _DCH), jnp.int32)]
            + [pltpu.VMEM((_DCH, D), jnp.float32)] * 2
            + [pltpu.SemaphoreType.DMA] * 4
        ),
    )


def _sc_dispatch(x2d, pos0, pos1):
    return _sc_dispatch_kernel()(x2d, pos0, pos1)


def _sc_combine_body(sh_hbm, y_hbm, p0_hbm, p1_hbm, w0_hbm, w1_hbm, out_hbm,
                     i0_v, i1_v, w0_v, w1_v, g0a, g1a, sha, g0b, g1b, shb,
                     sg0a, sg1a, ssha, sg0b, sg1b, sshb, sta, stb):
    wid = lax.axis_index("s") * _NC + lax.axis_index("c")
    base = wid * _TPW
    pltpu.sync_copy(p0_hbm.at[pl.ds(base, _TPW)], i0_v)
    pltpu.sync_copy(p1_hbm.at[pl.ds(base, _TPW)], i1_v)
    pltpu.sync_copy(w0_hbm.at[pl.ds(base, _TPW)], w0_v)
    pltpu.sync_copy(w1_hbm.at[pl.ds(base, _TPW)], w1_v)
    g0 = (g0a, g0b)
    g1 = (g1a, g1b)
    sh = (sha, shb)
    sg0 = (sg0a, sg0b)
    sg1 = (sg1a, sg1b)
    ssh = (ssha, sshb)
    st = (sta, stb)

    def start_in(c):
        b = c % 2
        d1 = pltpu.async_copy(
            y_hbm.at[i0_v.at[pl.ds(c * _CCH, _CCH)]], g0[b], sg0[b])
        d2 = pltpu.async_copy(
            y_hbm.at[i1_v.at[pl.ds(c * _CCH, _CCH)]], g1[b], sg1[b])
        d3 = pltpu.async_copy(
            sh_hbm.at[pl.ds(base + c * _CCH, _CCH)], sh[b], ssh[b])
        return (d1, d2, d3)

    ins = [None] * _CNCH
    outs = [None] * _CNCH
    ins[0] = start_in(0)
    for c in range(_CNCH):
        b = c % 2
        for dsc in ins[c]:
            dsc.wait()
        if c + 1 < _CNCH:
            if c >= 1:
                outs[c - 1].wait()
            ins[c + 1] = start_in(c + 1)

        wv0 = w0_v[pl.ds(c * _CCH, _CCH)]
        wv1 = w1_v[pl.ds(c * _CCH, _CCH)]

        def col(jj, carry, _b=b, _w0=wv0, _w1=wv1):
            sl = pl.ds(jj * 16, 16)
            for r in range(_CCH):
                sh[_b][r, sl] = (sh[_b][r, sl] + _w0[r] * g0[_b][r, sl]
                                 + _w1[r] * g1[_b][r, sl])
            return carry

        lax.fori_loop(0, D // 16, col, 0)
        outs[c] = pltpu.async_copy(
            sh[b], out_hbm.at[pl.ds(base + c * _CCH, _CCH)], st[b])
    outs[_CNCH - 2].wait()
    outs[_CNCH - 1].wait()


@functools.lru_cache(maxsize=None)
def _sc_combine_kernel():
    return pl.kernel(
        _sc_combine_body,
        out_type=jax.ShapeDtypeStruct((N, D), jnp.float32),
        mesh=plsc.VectorSubcoreMesh(core_axis_name="c", subcore_axis_name="s"),
        scratch_types=(
            [pltpu.VMEM((_TPW,), jnp.int32),
             pltpu.VMEM((_TPW,), jnp.int32),
             pltpu.VMEM((_TPW,), jnp.float32),
             pltpu.VMEM((_TPW,), jnp.float32)]
            + [pltpu.VMEM((_CCH, D), jnp.float32)] * 6
            + [pltpu.SemaphoreType.DMA] * 8
        ),
    )


def _sc_combine(shared2d, y, pos0, pos1, w0, w1):
    return _sc_combine_kernel()(shared2d, y, pos0, pos1, w0, w1)


def kernel(x, sw1, sb1, sw2, sb2, rw1, rb1, rw2, rb2, gw, gb):
    x2d = x.reshape(N, D)
    gwp = jnp.pad(gw, ((0, 0), (0, GP - E)))
    gbp = jnp.pad(gb, (0, GP - E), constant_values=-1e30).reshape(1, GP)

    gate = _gate(x2d, gwp, gbp)
    w0 = gate[:, 0]
    w1 = gate[:, 1]
    i0 = gate[:, 2].astype(jnp.int32)
    i1 = gate[:, 3].astype(jnp.int32)

    e_flat = jnp.concatenate([i0, i1])
    oh = (e_flat[:, None] == jnp.arange(E, dtype=jnp.int32)[None, :]
          ).astype(jnp.float32)
    BK = 128
    NB = (K * N) // BK
    oh3 = oh.reshape(NB, BK, E)
    tri = jnp.tril(jnp.ones((BK, BK), jnp.float32))
    intra = jax.lax.dot_general(
        oh3, tri, (((1,), (1,)), ((), ())),
        preferred_element_type=jnp.float32).transpose(0, 2, 1)
    blk_tot = intra[:, -1, :]
    tri_x = jnp.tril(jnp.ones((NB, NB), jnp.float32), -1)
    blk_pref = jax.lax.dot_general(
        tri_x, blk_tot, (((1,), (0,)), ((), ())),
        preferred_element_type=jnp.float32)
    prefix = (intra + blk_pref[:, None, :]).reshape(K * N, E)
    seq = (jnp.sum(oh * prefix, axis=1) - 1.0).astype(jnp.int32)
    counts = prefix[-1].astype(jnp.int32)
    tiles_per_e = (counts + T - 1) // T
    tile_start = jnp.concatenate(
        [jnp.zeros((1,), jnp.int32),
         jnp.cumsum(tiles_per_e)[:-1].astype(jnp.int32)])
    start = tile_start * T
    pos = (jnp.sum(oh * start[None, :].astype(jnp.float32), axis=1)
           ).astype(jnp.int32) + seq
    pos0, pos1 = pos[:N], pos[N:]
    tile_expert = (jnp.sum(
        (tile_start[None, :] <= jnp.arange(NT, dtype=jnp.int32)[:, None])
        .astype(jnp.int32), axis=1) - 1).astype(jnp.int32)

    xd = _sc_dispatch(x2d, pos0, pos1)
    shared2d = _shared_ffn(
        x2d, sw1, sb1.reshape(1, 2 * FF), sw2, sb2.reshape(1, D))

    y = _grouped_ffn(xd, rw1, rb1, rw2, rb2, tile_expert)

    out2d = _sc_combine(shared2d, y, pos0, pos1, w0, w1)
    return out2d.reshape(B, S, D)
